# baseline (speedup 1.0000x reference)
import jax
import jax.numpy as jnp
from jax.experimental import pallas as pl
from jax.experimental.pallas import tpu as pltpu

_SEQ_BLOCK = 512


def _add_kernel(x_ref, pos_ref, o_ref):
    o_ref[...] = x_ref[...] + pos_ref[...][None]


def kernel(x, pos_table):
    batch, seq_len, hidden = x.shape
    pos = pos_table[:seq_len]
    sblocks = seq_len // _SEQ_BLOCK

    out = pl.pallas_call(
        _add_kernel,
        grid=(sblocks,),
        in_specs=[
            pl.BlockSpec((batch, _SEQ_BLOCK, hidden), lambda s: (0, s, 0)),
            pl.BlockSpec((_SEQ_BLOCK, hidden), lambda s: (s, 0)),
        ],
        out_specs=pl.BlockSpec((batch, _SEQ_BLOCK, hidden), lambda s: (0, s, 0)),
        out_shape=jax.ShapeDtypeStruct((batch, seq_len, hidden), x.dtype),
        compiler_params=pltpu.CompilerParams(
            dimension_semantics=("arbitrary",),
        ),
    )(x, pos)
    return out


# speedup vs baseline: 1.0062x; 1.0062x over previous
import jax
import jax.numpy as jnp
from jax.experimental import pallas as pl
from jax.experimental.pallas import tpu as pltpu

_SEQ_BLOCK = 2048


def _add_kernel(x_ref, pos_ref, o_ref):
    o_ref[...] = x_ref[...] + pos_ref[...]


def kernel(x, pos_table):
    batch, seq_len, hidden = x.shape
    pos = pos_table[:seq_len]
    sblocks = seq_len // _SEQ_BLOCK

    grid = (sblocks, batch)
    out = pl.pallas_call(
        _add_kernel,
        grid=grid,
        in_specs=[
            pl.BlockSpec((1, _SEQ_BLOCK, hidden), lambda s, b: (b, s, 0)),
            pl.BlockSpec((_SEQ_BLOCK, hidden), lambda s, b: (s, 0)),
        ],
        out_specs=pl.BlockSpec((1, _SEQ_BLOCK, hidden), lambda s, b: (b, s, 0)),
        out_shape=jax.ShapeDtypeStruct((batch, seq_len, hidden), x.dtype),
        compiler_params=pltpu.CompilerParams(
            dimension_semantics=("arbitrary", "arbitrary"),
        ),
    )(x, pos)
    return out
